# baseline (speedup 1.0000x reference)
import functools

import jax
import jax.numpy as jnp
from jax.experimental import pallas as pl
from jax.experimental.pallas import tpu as pltpu
from jax.experimental.pallas import tpu_sc as plsc

R_BLK = 256
LANES = 128
GATHER_WIN = 128


def _argmax_body(shift_ref, x_ref, tok_ref, idx_ref):
    c = x_ref.shape[2]
    n_chunks = c // LANES

    runmax = x_ref[0, :, 0:LANES]
    runcol = jnp.zeros((R_BLK, LANES), jnp.int32)
    for j in range(1, n_chunks):
        chunk = x_ref[0, :, j * LANES:(j + 1) * LANES]
        upd = chunk > runmax
        runmax = jnp.where(upd, chunk, runmax)
        runcol = jnp.where(upd, j, runcol)

    rowmax = jnp.max(runmax, axis=1, keepdims=True)
    lane = jax.lax.broadcasted_iota(jnp.int32, (R_BLK, LANES), 1)
    cfull = runcol * LANES + lane
    masked = jnp.where(runmax == rowmax, cfull, c)
    amax = jnp.min(masked, axis=1).reshape(1, R_BLK)
    tokens = amax + shift_ref[0]
    tok_ref[...] = tokens
    idx_ref[...] = jnp.clip(tokens, 0, c - 1)


def _argmax_tokens(pt, tokens_shift, b_lo, nb):
    _, hw, c = pt.shape
    n_rb = hw // R_BLK
    shift = jnp.asarray(tokens_shift, jnp.int32).reshape(1)

    tok, idx = pl.pallas_call(
        _argmax_body,
        grid=(nb, n_rb),
        in_specs=[
            pl.BlockSpec(memory_space=pltpu.SMEM),
            pl.BlockSpec((1, R_BLK, c), lambda i, r: (i + b_lo, r, 0)),
        ],
        out_specs=[
            pl.BlockSpec((1, R_BLK), lambda i, r: (0, i * n_rb + r)),
            pl.BlockSpec((1, R_BLK), lambda i, r: (0, i * n_rb + r)),
        ],
        out_shape=[
            jax.ShapeDtypeStruct((1, nb * hw), jnp.int32),
            jax.ShapeDtypeStruct((1, nb * hw), jnp.int32),
        ],
        compiler_params=pltpu.CompilerParams(
            dimension_semantics=("arbitrary", "arbitrary")
        ),
    )(shift, pt)
    return tok, idx


def _sc_gather(embeddings, idx_flat):
    n = idx_flat.shape[1]
    d = embeddings.shape[1]
    mesh = plsc.VectorSubcoreMesh(core_axis_name="core", subcore_axis_name="subcore")

    @pl.kernel(
        out_type=jax.ShapeDtypeStruct((n, d), embeddings.dtype),
        mesh=mesh,
    )
    def gk(e_hbm, i_hbm, o_hbm):
        def body(i_vmem, o_vmem):
            pltpu.sync_copy(e_hbm.at[i_vmem.at[0]], o_vmem)

        pltpu.emit_pipeline(
            body,
            grid=(n // GATHER_WIN,),
            in_specs=[pl.BlockSpec((1, GATHER_WIN), lambda i: (0, i))],
            out_specs=[pl.BlockSpec((GATHER_WIN, d), lambda i: (i, 0))],
            core_axis_name=("core", "subcore"),
            dimension_semantics=(pltpu.PARALLEL,),
        )(i_hbm, o_hbm)

    return gk(embeddings, idx_flat)


def kernel(probs, tokens_shift, embeddings):
    b, c, h, w = probs.shape
    hw = h * w
    d = embeddings.shape[1]

    pt = probs.transpose(0, 2, 3, 1).reshape(b, hw, c)
    half = b // 2
    toks, gs = [], []
    for s in range(2):
        tok, idx = _argmax_tokens(pt, tokens_shift, s * half, half)
        toks.append(tok)
        gs.append(_sc_gather(embeddings, idx))

    g = jnp.concatenate(gs, axis=0)
    tokens = jnp.concatenate(toks, axis=1)
    x = g.reshape(b, h, w, d).transpose(0, 3, 1, 2)
    return x, tokens.reshape(b, h, w)


# speedup vs baseline: 1.0882x; 1.0882x over previous
import functools

import jax
import jax.numpy as jnp
from jax.experimental import pallas as pl
from jax.experimental.pallas import tpu as pltpu
from jax.experimental.pallas import tpu_sc as plsc

R_BLK = 256
LANES = 128
GATHER_WIN = 128


def _argmax_body(shift_ref, x_ref, tok_ref, idx_ref, *, n_rb):
    c = x_ref.shape[2]
    n_chunks = c // LANES
    i = pl.program_id(0)
    r = pl.program_id(1)

    runmax = x_ref[0, :, 0:LANES]
    runcol = jnp.zeros((R_BLK, LANES), jnp.int32)
    for j in range(1, n_chunks):
        chunk = x_ref[0, :, j * LANES:(j + 1) * LANES]
        upd = chunk > runmax
        runmax = jnp.where(upd, chunk, runmax)
        runcol = jnp.where(upd, j, runcol)

    rowmax = jnp.max(runmax, axis=1, keepdims=True)
    lane = jax.lax.broadcasted_iota(jnp.int32, (R_BLK, LANES), 1)
    cfull = runcol * LANES + lane
    masked = jnp.where(runmax == rowmax, cfull, c)
    amax = jnp.min(masked, axis=1).reshape(1, R_BLK)
    tokens = amax + shift_ref[0]
    pos = (i * n_rb + r) * R_BLK
    tok_ref[0, pl.ds(pos, R_BLK)] = tokens[0]
    idx_ref[0, pl.ds(pos, R_BLK)] = jnp.clip(tokens[0], 0, c - 1)


def _argmax_tokens(pt, tokens_shift):
    b, hw, c = pt.shape
    n_rb = hw // R_BLK
    shift = jnp.asarray(tokens_shift, jnp.int32).reshape(1)

    tok, idx = pl.pallas_call(
        functools.partial(_argmax_body, n_rb=n_rb),
        grid=(b, n_rb),
        in_specs=[
            pl.BlockSpec(memory_space=pltpu.SMEM),
            pl.BlockSpec((1, R_BLK, c), lambda i, r: (i, r, 0)),
        ],
        out_specs=[
            pl.BlockSpec((1, b * hw), lambda i, r: (0, 0)),
            pl.BlockSpec((1, b * hw), lambda i, r: (0, 0)),
        ],
        out_shape=[
            jax.ShapeDtypeStruct((1, b * hw), jnp.int32),
            jax.ShapeDtypeStruct((1, b * hw), jnp.int32),
        ],
        compiler_params=pltpu.CompilerParams(
            dimension_semantics=("arbitrary", "arbitrary")
        ),
    )(shift, pt)
    return tok, idx


def _sc_gather(embeddings, idx_flat):
    n = idx_flat.shape[1]
    d = embeddings.shape[1]
    mesh = plsc.VectorSubcoreMesh(core_axis_name="core", subcore_axis_name="subcore")

    @pl.kernel(
        out_type=jax.ShapeDtypeStruct((n, d), embeddings.dtype),
        mesh=mesh,
    )
    def gk(e_hbm, i_hbm, o_hbm):
        def body(i_vmem, o_vmem):
            pltpu.sync_copy(e_hbm.at[i_vmem.at[0]], o_vmem)

        pltpu.emit_pipeline(
            body,
            grid=(n // GATHER_WIN,),
            in_specs=[pl.BlockSpec((1, GATHER_WIN), lambda i: (0, i))],
            out_specs=[pl.BlockSpec((GATHER_WIN, d), lambda i: (i, 0))],
            core_axis_name=("core", "subcore"),
            dimension_semantics=(pltpu.PARALLEL,),
        )(i_hbm, o_hbm)

    return gk(embeddings, idx_flat)


def kernel(probs, tokens_shift, embeddings):
    b, c, h, w = probs.shape
    hw = h * w
    d = embeddings.shape[1]

    pt = probs.transpose(0, 2, 3, 1).reshape(b, hw, c)
    tokens, idx = _argmax_tokens(pt, tokens_shift)
    g = _sc_gather(embeddings, idx)
    x = g.reshape(b, h, w, d).transpose(0, 3, 1, 2)
    return x, tokens.reshape(b, h, w)
